# Initial kernel scaffold; baseline (speedup 1.0000x reference)
import jax
import jax.numpy as jnp
from jax.experimental import pallas as pl

N = 10000
D = 128
M5 = 5
NU = 2500
ND = 7500
NU_P = 2560
ND_P = 7680
BD = 512
BN = 1000
EPS = 1e-5


def _feat_body(h_ref, slp_ref, wh_ref, wsp_ref, bemb_ref, w1_ref,
               feat_ref, fw1_ref):
    f = (jnp.dot(h_ref[...], wh_ref[...], preferred_element_type=jnp.float32)
         + jnp.dot(slp_ref[...], wsp_ref[...], preferred_element_type=jnp.float32)
         + bemb_ref[...])
    feat_ref[...] = f
    fw1_ref[...] = jnp.dot(f, w1_ref[...], preferred_element_type=jnp.float32)


def _features(h, slp, W_emb, b_emb, W1):
    wh = W_emb[:D, :]
    wsp = jnp.zeros((D, D), jnp.float32).at[:3, :].set(W_emb[D:, :])
    grid = N // BN
    return pl.pallas_call(
        _feat_body,
        grid=(grid,),
        in_specs=[
            pl.BlockSpec((BN, D), lambda i: (i, 0)),
            pl.BlockSpec((BN, D), lambda i: (i, 0)),
            pl.BlockSpec((D, D), lambda i: (0, 0)),
            pl.BlockSpec((D, D), lambda i: (0, 0)),
            pl.BlockSpec((1, D), lambda i: (0, 0)),
            pl.BlockSpec((D, D), lambda i: (0, 0)),
        ],
        out_specs=[
            pl.BlockSpec((BN, D), lambda i: (i, 0)),
            pl.BlockSpec((BN, D), lambda i: (i, 0)),
        ],
        out_shape=[
            jax.ShapeDtypeStruct((N, D), jnp.float32),
            jax.ShapeDtypeStruct((N, D), jnp.float32),
        ],
    )(h, slp, wh, wsp, b_emb.reshape(1, D), W1)


def _knn_body(sdp_ref, fwd_ref, sut_ref, fwu_ref, b1_ref, w2_ref, b2_ref,
              acc_ref):
    pid = pl.program_id(0)

    @pl.when(pid == 0)
    def _init():
        acc_ref[...] = jnp.zeros_like(acc_ref)
        acc_ref[2:3, :] = jnp.full((1, NU_P), 2.0, jnp.float32)

    sd0 = sdp_ref[:, 0:1]
    sd1 = sdp_ref[:, 1:2]
    sd2c = sdp_ref[:, 2:3]
    su0 = sut_ref[0:1, :]
    su1 = sut_ref[1:2, :]
    su2c = sut_ref[2:3, :]
    sdq = sd0 * sd0 + sd1 * sd1 + sd2c * sd2c
    suq = su0 * su0 + su1 * su1 + su2c * su2c
    cross = sd0 * su0 + sd1 * su1 + sd2c * su2c
    d2 = sdq + suq - 2.0 * cross

    row = pl.program_id(0) * BD + jax.lax.broadcasted_iota(jnp.int32, (BD, 1), 0)
    validf = jnp.where(row < ND, 1.0, 0.0)
    lane = jax.lax.broadcasted_iota(jnp.float32, (BD, NU_P), 1)

    fwd = fwd_ref[...]
    b1 = b1_ref[...]
    w2 = w2_ref[...]
    b2 = b2_ref[...]

    s_acc = jnp.zeros((1, NU_P), jnp.float32)
    mx_acc = jnp.zeros((1, NU_P), jnp.float32)
    mn_acc = jnp.full((1, NU_P), 2.0, jnp.float32)
    dg_acc = jnp.zeros((1, NU_P), jnp.float32)

    for _ in range(M5):
        v = jnp.min(d2, axis=1, keepdims=True)
        idxv = jnp.min(jnp.where(d2 == v, lane, 4096.0), axis=1, keepdims=True)
        sel = lane == idxv
        d2 = jnp.where(sel, 1e30, d2)
        maskf = jnp.where(sel, 1.0, 0.0) * validf
        g = jnp.dot(maskf, fwu_ref[...], preferred_element_type=jnp.float32)
        t = jnp.maximum(fwd - g + b1, 0.0)
        att = jnp.maximum(jnp.dot(t, w2, preferred_element_type=jnp.float32)
                          + b2, 0.0)
        c = jnp.exp(-att)
        w = maskf * c
        s_acc = s_acc + jnp.sum(w, axis=0, keepdims=True)
        mx_acc = jnp.maximum(mx_acc, jnp.max(w, axis=0, keepdims=True))
        mn_acc = jnp.minimum(mn_acc, jnp.min(w + 2.0 * (1.0 - maskf), axis=0,
                                             keepdims=True))
        dg_acc = dg_acc + jnp.sum(maskf, axis=0, keepdims=True)

    acc_ref[0:1, :] += s_acc
    acc_ref[1:2, :] = jnp.maximum(acc_ref[1:2, :], mx_acc)
    acc_ref[2:3, :] = jnp.minimum(acc_ref[2:3, :], mn_acc)
    acc_ref[3:4, :] += dg_acc


def _knn_agg(sdp, fwd, sut, fwu, b1, W2, b2):
    grid = ND_P // BD
    return pl.pallas_call(
        _knn_body,
        grid=(grid,),
        in_specs=[
            pl.BlockSpec((BD, D), lambda i: (i, 0)),
            pl.BlockSpec((BD, D), lambda i: (i, 0)),
            pl.BlockSpec((8, NU_P), lambda i: (0, 0)),
            pl.BlockSpec((NU_P, D), lambda i: (0, 0)),
            pl.BlockSpec((1, D), lambda i: (0, 0)),
            pl.BlockSpec((D, 1), lambda i: (0, 0)),
            pl.BlockSpec((1, 1), lambda i: (0, 0)),
        ],
        out_specs=pl.BlockSpec((8, NU_P), lambda i: (0, 0)),
        out_shape=jax.ShapeDtypeStruct((8, NU_P), jnp.float32),
    )(sdp, fwd, sut, fwu, b1.reshape(1, D), W2, b2.reshape(1, 1))


def _agg_body(f_ref, s_ref, mx_ref, mn_ref, dg_ref, wo_ref, bo_ref,
              x1_ref, st_ref):
    pid = pl.program_id(0)

    @pl.when(pid == 0)
    def _init():
        st_ref[...] = jnp.zeros_like(st_ref)

    f = f_ref[...]
    s = s_ref[...]
    mx = mx_ref[...]
    mn = mn_ref[...]
    dg = dg_ref[...]
    mean_agg = f * (s / jnp.maximum(dg, 1.0))
    max_agg = f * jnp.where(f >= 0.0, mx, mn)
    cat = jnp.concatenate([mean_agg, max_agg], axis=1)
    agg = jnp.dot(cat, wo_ref[...], preferred_element_type=jnp.float32) \
        + bo_ref[...]
    agg = jnp.where(dg > 0.0, agg, 0.0)
    x1 = f + agg
    x1_ref[...] = x1
    st_ref[0:1, :] += jnp.sum(x1, axis=0, keepdims=True)
    st_ref[1:2, :] += jnp.sum(x1 * x1, axis=0, keepdims=True)


def _agg_stage(feat, s_g, mx_g, mn_g, dg_g, WO, bO):
    grid = N // BN
    return pl.pallas_call(
        _agg_body,
        grid=(grid,),
        in_specs=[
            pl.BlockSpec((BN, D), lambda i: (i, 0)),
            pl.BlockSpec((BN, 1), lambda i: (i, 0)),
            pl.BlockSpec((BN, 1), lambda i: (i, 0)),
            pl.BlockSpec((BN, 1), lambda i: (i, 0)),
            pl.BlockSpec((BN, 1), lambda i: (i, 0)),
            pl.BlockSpec((2 * D, D), lambda i: (0, 0)),
            pl.BlockSpec((1, D), lambda i: (0, 0)),
        ],
        out_specs=[
            pl.BlockSpec((BN, D), lambda i: (i, 0)),
            pl.BlockSpec((8, D), lambda i: (0, 0)),
        ],
        out_shape=[
            jax.ShapeDtypeStruct((N, D), jnp.float32),
            jax.ShapeDtypeStruct((8, D), jnp.float32),
        ],
    )(feat, s_g, mx_g, mn_g, dg_g, WO, bO.reshape(1, D))


def _ffn_body(x1_ref, st_ref, g1_ref, be1_ref, wf1_ref, bf1_ref, wf2_ref,
              bf2_ref, x2_ref, st2_ref):
    pid = pl.program_id(0)

    @pl.when(pid == 0)
    def _init():
        st2_ref[...] = jnp.zeros_like(st2_ref)

    mu = st_ref[0:1, :] / N
    var = st_ref[1:2, :] / N - mu * mu
    hh = (x1_ref[...] - mu) / jnp.sqrt(var + EPS) * g1_ref[...] + be1_ref[...]
    t = jnp.maximum(jnp.dot(hh, wf1_ref[...],
                            preferred_element_type=jnp.float32)
                    + bf1_ref[...], 0.0)
    y = jnp.dot(t, wf2_ref[...], preferred_element_type=jnp.float32) \
        + bf2_ref[...]
    x2 = hh + y
    x2_ref[...] = x2
    st2_ref[0:1, :] += jnp.sum(x2, axis=0, keepdims=True)
    st2_ref[1:2, :] += jnp.sum(x2 * x2, axis=0, keepdims=True)


def _ffn_stage(x1, st1, g1, be1, Wf1, bf1, Wf2, bf2):
    grid = N // BN
    return pl.pallas_call(
        _ffn_body,
        grid=(grid,),
        in_specs=[
            pl.BlockSpec((BN, D), lambda i: (i, 0)),
            pl.BlockSpec((8, D), lambda i: (0, 0)),
            pl.BlockSpec((1, D), lambda i: (0, 0)),
            pl.BlockSpec((1, D), lambda i: (0, 0)),
            pl.BlockSpec((D, 2 * D), lambda i: (0, 0)),
            pl.BlockSpec((1, 2 * D), lambda i: (0, 0)),
            pl.BlockSpec((2 * D, D), lambda i: (0, 0)),
            pl.BlockSpec((1, D), lambda i: (0, 0)),
        ],
        out_specs=[
            pl.BlockSpec((BN, D), lambda i: (i, 0)),
            pl.BlockSpec((8, D), lambda i: (0, 0)),
        ],
        out_shape=[
            jax.ShapeDtypeStruct((N, D), jnp.float32),
            jax.ShapeDtypeStruct((8, D), jnp.float32),
        ],
    )(x1, st1, g1.reshape(1, D), be1.reshape(1, D), Wf1,
      bf1.reshape(1, 2 * D), Wf2, bf2.reshape(1, D))


def _bn2_body(x2_ref, st_ref, g2_ref, be2_ref, out_ref):
    mu = st_ref[0:1, :] / N
    var = st_ref[1:2, :] / N - mu * mu
    out_ref[...] = (x2_ref[...] - mu) / jnp.sqrt(var + EPS) * g2_ref[...] \
        + be2_ref[...]


def _bn2_stage(x2, st2, g2, be2):
    grid = N // BN
    return pl.pallas_call(
        _bn2_body,
        grid=(grid,),
        in_specs=[
            pl.BlockSpec((BN, D), lambda i: (i, 0)),
            pl.BlockSpec((8, D), lambda i: (0, 0)),
            pl.BlockSpec((1, D), lambda i: (0, 0)),
            pl.BlockSpec((1, D), lambda i: (0, 0)),
        ],
        out_specs=pl.BlockSpec((BN, D), lambda i: (i, 0)),
        out_shape=jax.ShapeDtypeStruct((N, D), jnp.float32),
    )(x2, st2, g2.reshape(1, D), be2.reshape(1, D))


def kernel(h, s_l, scores, W_emb, b_emb, W1, b1, W2, b2, WO, bO, g1, be1,
           Wf1, bf1, Wf2, bf2, g2, be2):
    order = jnp.flip(jnp.argsort(scores))
    nodes_up = jnp.sort(order[:NU])
    nodes_down = jnp.sort(order[NU:])

    slp = jnp.zeros((N, D), jnp.float32).at[:, :3].set(s_l)

    feat, fw1 = _features(h, slp, W_emb, b_emb, W1)

    sut = jnp.full((8, NU_P), 1e6, jnp.float32)
    sut = sut.at[:3, :NU].set(jnp.take(s_l, nodes_up, axis=0).T)
    fwu = jnp.pad(jnp.take(fw1, nodes_up, axis=0), ((0, NU_P - NU), (0, 0)))
    sdp = jnp.pad(jnp.take(slp, nodes_down, axis=0),
                  ((0, ND_P - ND), (0, 0)))
    fwd = jnp.pad(jnp.take(fw1, nodes_down, axis=0),
                  ((0, ND_P - ND), (0, 0)))

    acc = _knn_agg(sdp, fwd, sut, fwu, b1, W2, b2)

    zero = jnp.zeros((N,), jnp.float32)
    s_g = zero.at[nodes_up].set(acc[0, :NU]).reshape(N, 1)
    mx_g = zero.at[nodes_up].set(acc[1, :NU]).reshape(N, 1)
    mn_g = zero.at[nodes_up].set(acc[2, :NU]).reshape(N, 1)
    dg_g = zero.at[nodes_up].set(acc[3, :NU]).reshape(N, 1)

    x1, st1 = _agg_stage(feat, s_g, mx_g, mn_g, dg_g, WO, bO)
    x2, st2 = _ffn_stage(x1, st1, g1, be1, Wf1, bf1, Wf2, bf2)
    return _bn2_stage(x2, st2, g2, be2)


# speedup vs baseline: 2.8889x; 2.8889x over previous
import jax
import jax.numpy as jnp
from jax.experimental import pallas as pl

N = 10000
D = 128
M5 = 5
NU = 2500
ND = 7500
NU_P = 2560
ND_P = 7680
BD = 512
BN = 1000
EPS = 1e-5


def _feat_body(h_ref, slp_ref, wh_ref, wsp_ref, bemb_ref, w1_ref,
               feat_ref, fw1_ref):
    f = (jnp.dot(h_ref[...], wh_ref[...], preferred_element_type=jnp.float32)
         + jnp.dot(slp_ref[...], wsp_ref[...], preferred_element_type=jnp.float32)
         + bemb_ref[...])
    feat_ref[...] = f
    fw1_ref[...] = jnp.dot(f, w1_ref[...], preferred_element_type=jnp.float32)


def _features(h, slp, W_emb, b_emb, W1):
    wh = W_emb[:D, :]
    wsp = jnp.zeros((D, D), jnp.float32).at[:3, :].set(W_emb[D:, :])
    grid = N // BN
    return pl.pallas_call(
        _feat_body,
        grid=(grid,),
        in_specs=[
            pl.BlockSpec((BN, D), lambda i: (i, 0)),
            pl.BlockSpec((BN, D), lambda i: (i, 0)),
            pl.BlockSpec((D, D), lambda i: (0, 0)),
            pl.BlockSpec((D, D), lambda i: (0, 0)),
            pl.BlockSpec((1, D), lambda i: (0, 0)),
            pl.BlockSpec((D, D), lambda i: (0, 0)),
        ],
        out_specs=[
            pl.BlockSpec((BN, D), lambda i: (i, 0)),
            pl.BlockSpec((BN, D), lambda i: (i, 0)),
        ],
        out_shape=[
            jax.ShapeDtypeStruct((N, D), jnp.float32),
            jax.ShapeDtypeStruct((N, D), jnp.float32),
        ],
    )(h, slp, wh, wsp, b_emb.reshape(1, D), W1)


def _knn_body(sdp_ref, fwd_ref, sut_ref, fwu_ref, b1_ref, w2_ref, b2_ref,
              acc_ref):
    pid = pl.program_id(0)

    @pl.when(pid == 0)
    def _init():
        acc_ref[...] = jnp.zeros_like(acc_ref)
        acc_ref[2:3, :] = jnp.full((1, NU_P), 2.0, jnp.float32)

    sd0 = sdp_ref[:, 0:1]
    sd1 = sdp_ref[:, 1:2]
    sd2c = sdp_ref[:, 2:3]
    su0 = sut_ref[0:1, :]
    su1 = sut_ref[1:2, :]
    su2c = sut_ref[2:3, :]
    sdq = sd0 * sd0 + sd1 * sd1 + sd2c * sd2c
    suq = su0 * su0 + su1 * su1 + su2c * su2c
    cross = jnp.dot(sdp_ref[:, 0:8], sut_ref[...],
                    preferred_element_type=jnp.float32)
    d2 = sdq + suq - 2.0 * cross

    row = pl.program_id(0) * BD + jax.lax.broadcasted_iota(jnp.int32, (BD, 1), 0)
    validf = jnp.where(row < ND, 1.0, 0.0)
    lane = jax.lax.broadcasted_iota(jnp.int32, (BD, NU_P), 1).astype(jnp.float32)

    fwd = fwd_ref[...]
    b1 = b1_ref[...]
    w2 = w2_ref[...]
    b2 = b2_ref[...]

    s_acc = jnp.zeros((1, NU_P), jnp.float32)
    mx_acc = jnp.zeros((1, NU_P), jnp.float32)
    mn_acc = jnp.full((1, NU_P), 2.0, jnp.float32)
    dg_acc = jnp.zeros((1, NU_P), jnp.float32)

    for _ in range(M5):
        v = jnp.min(d2, axis=1, keepdims=True)
        idxv = jnp.min(jnp.where(d2 == v, lane, 4096.0), axis=1, keepdims=True)
        sel = lane == idxv
        d2 = jnp.where(sel, 1e30, d2)
        maskf = jnp.where(sel, 1.0, 0.0) * validf
        g = jnp.dot(maskf, fwu_ref[...], preferred_element_type=jnp.float32)
        t = jnp.maximum(fwd - g + b1, 0.0)
        att = jnp.maximum(jnp.dot(t, w2, preferred_element_type=jnp.float32)
                          + b2, 0.0)
        c = jnp.exp(-att)
        w = maskf * c
        s_acc = s_acc + jnp.sum(w, axis=0, keepdims=True)
        mx_acc = jnp.maximum(mx_acc, jnp.max(w, axis=0, keepdims=True))
        mn_acc = jnp.minimum(mn_acc, jnp.min(w + 2.0 * (1.0 - maskf), axis=0,
                                             keepdims=True))
        dg_acc = dg_acc + jnp.sum(maskf, axis=0, keepdims=True)

    acc_ref[0:1, :] += s_acc
    acc_ref[1:2, :] = jnp.maximum(acc_ref[1:2, :], mx_acc)
    acc_ref[2:3, :] = jnp.minimum(acc_ref[2:3, :], mn_acc)
    acc_ref[3:4, :] += dg_acc


def _knn_agg(sdp, fwd, sut, fwu, b1, W2, b2):
    grid = ND_P // BD
    return pl.pallas_call(
        _knn_body,
        grid=(grid,),
        in_specs=[
            pl.BlockSpec((BD, D), lambda i: (i, 0)),
            pl.BlockSpec((BD, D), lambda i: (i, 0)),
            pl.BlockSpec((8, NU_P), lambda i: (0, 0)),
            pl.BlockSpec((NU_P, D), lambda i: (0, 0)),
            pl.BlockSpec((1, D), lambda i: (0, 0)),
            pl.BlockSpec((D, 1), lambda i: (0, 0)),
            pl.BlockSpec((1, 1), lambda i: (0, 0)),
        ],
        out_specs=pl.BlockSpec((8, NU_P), lambda i: (0, 0)),
        out_shape=jax.ShapeDtypeStruct((8, NU_P), jnp.float32),
    )(sdp, fwd, sut, fwu, b1.reshape(1, D), W2, b2.reshape(1, 1))


def _agg_body(f_ref, s_ref, mx_ref, mn_ref, dg_ref, wo_ref, bo_ref,
              x1_ref, st_ref):
    pid = pl.program_id(0)

    @pl.when(pid == 0)
    def _init():
        st_ref[...] = jnp.zeros_like(st_ref)

    f = f_ref[...]
    s = s_ref[...]
    mx = mx_ref[...]
    mn = mn_ref[...]
    dg = dg_ref[...]
    mean_agg = f * (s / jnp.maximum(dg, 1.0))
    max_agg = f * jnp.where(f >= 0.0, mx, mn)
    cat = jnp.concatenate([mean_agg, max_agg], axis=1)
    agg = jnp.dot(cat, wo_ref[...], preferred_element_type=jnp.float32) \
        + bo_ref[...]
    agg = jnp.where(dg > 0.0, agg, 0.0)
    x1 = f + agg
    x1_ref[...] = x1
    st_ref[0:1, :] += jnp.sum(x1, axis=0, keepdims=True)
    st_ref[1:2, :] += jnp.sum(x1 * x1, axis=0, keepdims=True)


def _agg_stage(feat, s_g, mx_g, mn_g, dg_g, WO, bO):
    grid = N // BN
    return pl.pallas_call(
        _agg_body,
        grid=(grid,),
        in_specs=[
            pl.BlockSpec((BN, D), lambda i: (i, 0)),
            pl.BlockSpec((BN, 1), lambda i: (i, 0)),
            pl.BlockSpec((BN, 1), lambda i: (i, 0)),
            pl.BlockSpec((BN, 1), lambda i: (i, 0)),
            pl.BlockSpec((BN, 1), lambda i: (i, 0)),
            pl.BlockSpec((2 * D, D), lambda i: (0, 0)),
            pl.BlockSpec((1, D), lambda i: (0, 0)),
        ],
        out_specs=[
            pl.BlockSpec((BN, D), lambda i: (i, 0)),
            pl.BlockSpec((8, D), lambda i: (0, 0)),
        ],
        out_shape=[
            jax.ShapeDtypeStruct((N, D), jnp.float32),
            jax.ShapeDtypeStruct((8, D), jnp.float32),
        ],
    )(feat, s_g, mx_g, mn_g, dg_g, WO, bO.reshape(1, D))


def _ffn_body(x1_ref, st_ref, g1_ref, be1_ref, wf1_ref, bf1_ref, wf2_ref,
              bf2_ref, x2_ref, st2_ref):
    pid = pl.program_id(0)

    @pl.when(pid == 0)
    def _init():
        st2_ref[...] = jnp.zeros_like(st2_ref)

    mu = st_ref[0:1, :] / N
    var = st_ref[1:2, :] / N - mu * mu
    hh = (x1_ref[...] - mu) / jnp.sqrt(var + EPS) * g1_ref[...] + be1_ref[...]
    t = jnp.maximum(jnp.dot(hh, wf1_ref[...],
                            preferred_element_type=jnp.float32)
                    + bf1_ref[...], 0.0)
    y = jnp.dot(t, wf2_ref[...], preferred_element_type=jnp.float32) \
        + bf2_ref[...]
    x2 = hh + y
    x2_ref[...] = x2
    st2_ref[0:1, :] += jnp.sum(x2, axis=0, keepdims=True)
    st2_ref[1:2, :] += jnp.sum(x2 * x2, axis=0, keepdims=True)


def _ffn_stage(x1, st1, g1, be1, Wf1, bf1, Wf2, bf2):
    grid = N // BN
    return pl.pallas_call(
        _ffn_body,
        grid=(grid,),
        in_specs=[
            pl.BlockSpec((BN, D), lambda i: (i, 0)),
            pl.BlockSpec((8, D), lambda i: (0, 0)),
            pl.BlockSpec((1, D), lambda i: (0, 0)),
            pl.BlockSpec((1, D), lambda i: (0, 0)),
            pl.BlockSpec((D, 2 * D), lambda i: (0, 0)),
            pl.BlockSpec((1, 2 * D), lambda i: (0, 0)),
            pl.BlockSpec((2 * D, D), lambda i: (0, 0)),
            pl.BlockSpec((1, D), lambda i: (0, 0)),
        ],
        out_specs=[
            pl.BlockSpec((BN, D), lambda i: (i, 0)),
            pl.BlockSpec((8, D), lambda i: (0, 0)),
        ],
        out_shape=[
            jax.ShapeDtypeStruct((N, D), jnp.float32),
            jax.ShapeDtypeStruct((8, D), jnp.float32),
        ],
    )(x1, st1, g1.reshape(1, D), be1.reshape(1, D), Wf1,
      bf1.reshape(1, 2 * D), Wf2, bf2.reshape(1, D))


def _bn2_body(x2_ref, st_ref, g2_ref, be2_ref, out_ref):
    mu = st_ref[0:1, :] / N
    var = st_ref[1:2, :] / N - mu * mu
    out_ref[...] = (x2_ref[...] - mu) / jnp.sqrt(var + EPS) * g2_ref[...] \
        + be2_ref[...]


def _bn2_stage(x2, st2, g2, be2):
    grid = N // BN
    return pl.pallas_call(
        _bn2_body,
        grid=(grid,),
        in_specs=[
            pl.BlockSpec((BN, D), lambda i: (i, 0)),
            pl.BlockSpec((8, D), lambda i: (0, 0)),
            pl.BlockSpec((1, D), lambda i: (0, 0)),
            pl.BlockSpec((1, D), lambda i: (0, 0)),
        ],
        out_specs=pl.BlockSpec((BN, D), lambda i: (i, 0)),
        out_shape=jax.ShapeDtypeStruct((N, D), jnp.float32),
    )(x2, st2, g2.reshape(1, D), be2.reshape(1, D))


def kernel(h, s_l, scores, W_emb, b_emb, W1, b1, W2, b2, WO, bO, g1, be1,
           Wf1, bf1, Wf2, bf2, g2, be2):
    order = jnp.flip(jnp.argsort(scores))
    nodes_up = jnp.sort(order[:NU])
    nodes_down = jnp.sort(order[NU:])

    slp = jnp.zeros((N, D), jnp.float32).at[:, :3].set(s_l)

    feat, fw1 = _features(h, slp, W_emb, b_emb, W1)

    sut = jnp.zeros((8, NU_P), jnp.float32)
    sut = sut.at[:3, :].set(1e6)
    sut = sut.at[:3, :NU].set(jnp.take(s_l, nodes_up, axis=0).T)
    fwu = jnp.pad(jnp.take(fw1, nodes_up, axis=0), ((0, NU_P - NU), (0, 0)))
    sdp = jnp.pad(jnp.take(slp, nodes_down, axis=0),
                  ((0, ND_P - ND), (0, 0)))
    fwd = jnp.pad(jnp.take(fw1, nodes_down, axis=0),
                  ((0, ND_P - ND), (0, 0)))

    acc = _knn_agg(sdp, fwd, sut, fwu, b1, W2, b2)

    zero = jnp.zeros((N,), jnp.float32)
    s_g = zero.at[nodes_up].set(acc[0, :NU]).reshape(N, 1)
    mx_g = zero.at[nodes_up].set(acc[1, :NU]).reshape(N, 1)
    mn_g = zero.at[nodes_up].set(acc[2, :NU]).reshape(N, 1)
    dg_g = zero.at[nodes_up].set(acc[3, :NU]).reshape(N, 1)

    x1, st1 = _agg_stage(feat, s_g, mx_g, mn_g, dg_g, WO, bO)
    x2, st2 = _ffn_stage(x1, st1, g1, be1, Wf1, bf1, Wf2, bf2)
    return _bn2_stage(x2, st2, g2, be2)
